# Initial kernel scaffold; baseline (speedup 1.0000x reference)
import jax
import jax.numpy as jnp
from jax.experimental import pallas as pl


def kernel(text_token, table, W, b):
    raise NotImplementedError("write your pallas kernel here")


# speedup vs baseline: 28.2108x; 28.2108x over previous
import functools

import jax
import jax.numpy as jnp
from jax import lax
from jax.experimental import pallas as pl
from jax.experimental.pallas import tpu as pltpu
from jax.experimental.pallas import tpu_sc as plsc

_VOCAB = 18440
_EMBED = 100
_OUT = 2
_B = 4096
_L = 200

_NW = 32
_RPW = _B // _NW
_GPW = _RPW // 16


def _tw_body(t_ref, w_ref, o_ref):
    o_ref[...] = jnp.dot(
        t_ref[...], w_ref[...], preferred_element_type=jnp.float32
    ) * (1.0 / _L)


def _fold_table(table, W):
    return pl.pallas_call(
        _tw_body,
        out_shape=jax.ShapeDtypeStruct((_VOCAB, _OUT), jnp.float32),
    )(table, W)


@functools.partial(
    pl.kernel,
    out_type=jax.ShapeDtypeStruct((_NW, _OUT, _RPW), jnp.float32),
    mesh=plsc.VectorSubcoreMesh(core_axis_name="c", subcore_axis_name="s"),
    compiler_params=pltpu.CompilerParams(needs_layout_passes=False),
    scratch_types=[
        pltpu.VMEM((_GPW * _L * 16,), jnp.int32),
        pltpu.VMEM((_VOCAB,), jnp.float32),
        pltpu.VMEM((_VOCAB,), jnp.float32),
        pltpu.VMEM((_OUT * 16,), jnp.float32),
        pltpu.VMEM((_OUT, _RPW), jnp.float32),
    ],
)
def _sc_pool(tok_hbm, tw0_hbm, tw1_hbm, bias_hbm, out_hbm,
             tok_v, tw0_v, tw1_v, bias_v, out_v):
    wid = lax.axis_index("s") * 2 + lax.axis_index("c")
    ntok = _GPW * _L * 16
    pltpu.sync_copy(tok_hbm.at[pl.ds(wid * ntok, ntok)], tok_v)
    pltpu.sync_copy(tw0_hbm, tw0_v)
    pltpu.sync_copy(tw1_hbm, tw1_v)
    pltpu.sync_copy(bias_hbm, bias_v)
    bv0 = bias_v[pl.ds(0, 16)]
    bv1 = bias_v[pl.ds(16, 16)]

    for g in range(_GPW):
        def body(l, carry, g=g):
            a0, a1 = carry
            tok16 = tok_v[pl.ds(g * (_L * 16) + l * 16, 16)]
            v0 = plsc.load_gather(tw0_v, [tok16])
            v1 = plsc.load_gather(tw1_v, [tok16])
            return a0 + v0, a1 + v1

        zero = jnp.zeros((16,), jnp.float32)
        a0, a1 = lax.fori_loop(0, _L, body, (zero, zero))
        out_v[0, pl.ds(g * 16, 16)] = a0 + bv0
        out_v[1, pl.ds(g * 16, 16)] = a1 + bv1

    pltpu.sync_copy(out_v, out_hbm.at[wid])


def kernel(text_token, table, W, b):
    tw = _fold_table(table, W)
    tw0 = tw[:, 0]
    tw1 = tw[:, 1]
    tok_t = jnp.transpose(
        text_token.reshape(_B // 16, 16, _L), (0, 2, 1)
    ).reshape(-1)
    bias16 = jnp.broadcast_to(b[:, None], (_OUT, 16)).reshape(-1)
    out = _sc_pool(tok_t, tw0, tw1, bias16)
    return jnp.transpose(out, (0, 2, 1)).reshape(_B, _OUT)


# speedup vs baseline: 30.0898x; 1.0666x over previous
import functools

import jax
import jax.numpy as jnp
from jax import lax
from jax.experimental import pallas as pl
from jax.experimental.pallas import tpu as pltpu
from jax.experimental.pallas import tpu_sc as plsc

_VOCAB = 18440
_EMBED = 100
_OUT = 2
_B = 4096
_L = 200

_NW = 32
_RPW = _B // _NW
_GPW = _RPW // 16


def _tw_body(t_ref, w_ref, o_ref):
    o_ref[...] = jnp.dot(
        t_ref[...], w_ref[...], preferred_element_type=jnp.float32
    ) * (1.0 / _L)


def _fold_table(table, W):
    return pl.pallas_call(
        _tw_body,
        out_shape=jax.ShapeDtypeStruct((_VOCAB, _OUT), jnp.float32),
    )(table, W)


@functools.partial(
    pl.kernel,
    out_type=jax.ShapeDtypeStruct((_NW, _OUT, _RPW), jnp.float32),
    mesh=plsc.VectorSubcoreMesh(core_axis_name="c", subcore_axis_name="s"),
    compiler_params=pltpu.CompilerParams(needs_layout_passes=False),
    scratch_types=[
        pltpu.VMEM((_GPW * _L * 16,), jnp.int32),
        pltpu.VMEM((_VOCAB,), jnp.float32),
        pltpu.VMEM((_VOCAB,), jnp.float32),
        pltpu.VMEM((_OUT * 16,), jnp.float32),
        pltpu.VMEM((_OUT, _RPW), jnp.float32),
    ],
)
def _sc_pool(tok_hbm, tw0_hbm, tw1_hbm, bias_hbm, out_hbm,
             tok_v, tw0_v, tw1_v, bias_v, out_v):
    wid = lax.axis_index("s") * 2 + lax.axis_index("c")
    ntok = _GPW * _L * 16
    pltpu.sync_copy(tok_hbm.at[pl.ds(wid * ntok, ntok)], tok_v)
    pltpu.sync_copy(tw0_hbm, tw0_v)
    pltpu.sync_copy(tw1_hbm, tw1_v)
    pltpu.sync_copy(bias_hbm, bias_v)
    bv0 = bias_v[pl.ds(0, 16)]
    bv1 = bias_v[pl.ds(16, 16)]

    def body(l, carry):
        new = []
        off = l * 16
        for g in range(_GPW):
            a0, a1 = carry[2 * g], carry[2 * g + 1]
            tok16 = tok_v[pl.ds(off + g * (_L * 16), 16)]
            v0 = plsc.load_gather(tw0_v, [tok16])
            v1 = plsc.load_gather(tw1_v, [tok16])
            new.append(a0 + v0)
            new.append(a1 + v1)
        return tuple(new)

    zero = jnp.zeros((16,), jnp.float32)
    accs = lax.fori_loop(0, _L, body, (zero,) * (2 * _GPW))
    for g in range(_GPW):
        out_v[0, pl.ds(g * 16, 16)] = accs[2 * g] + bv0
        out_v[1, pl.ds(g * 16, 16)] = accs[2 * g + 1] + bv1

    pltpu.sync_copy(out_v, out_hbm.at[wid])


def kernel(text_token, table, W, b):
    tw = _fold_table(table, W)
    tw0 = tw[:, 0]
    tw1 = tw[:, 1]
    tok_t = jnp.transpose(
        text_token.reshape(_B // 16, 16, _L), (0, 2, 1)
    ).reshape(-1)
    bias16 = jnp.broadcast_to(b[:, None], (_OUT, 16)).reshape(-1)
    out = _sc_pool(tok_t, tw0, tw1, bias16)
    return jnp.transpose(out, (0, 2, 1)).reshape(_B, _OUT)


# speedup vs baseline: 48.2999x; 1.6052x over previous
import functools

import jax
import jax.numpy as jnp
from jax import lax
from jax.experimental import pallas as pl
from jax.experimental.pallas import tpu as pltpu
from jax.experimental.pallas import tpu_sc as plsc

_VOCAB = 18440
_EMBED = 100
_OUT = 2
_B = 4096
_L = 200

_NW = 32
_RPW = _B // _NW
_GPW = _RPW // 16


def _tw_body(wt_ref, t_ref, o_ref):
    o_ref[...] = lax.dot_general(
        wt_ref[...], t_ref[...],
        (((1,), (1,)), ((), ())),
        preferred_element_type=jnp.float32,
    ) * (1.0 / _L)


def _fold_table(table, Wt):
    return pl.pallas_call(
        _tw_body,
        out_shape=jax.ShapeDtypeStruct((_OUT, _VOCAB), jnp.float32),
    )(Wt, table)


@functools.partial(
    pl.kernel,
    out_type=jax.ShapeDtypeStruct((_NW, _OUT, _RPW), jnp.float32),
    mesh=plsc.VectorSubcoreMesh(core_axis_name="c", subcore_axis_name="s"),
    compiler_params=pltpu.CompilerParams(needs_layout_passes=False),
    scratch_types=[
        pltpu.VMEM((_RPW * _L,), jnp.int32),
        pltpu.VMEM((_VOCAB,), jnp.float32),
        pltpu.VMEM((_VOCAB,), jnp.float32),
        pltpu.VMEM((_OUT * 16,), jnp.float32),
        pltpu.VMEM((_OUT, _RPW), jnp.float32),
    ],
)
def _sc_pool(tok_hbm, tw0_hbm, tw1_hbm, bias_hbm, out_hbm,
             tok_v, tw0_v, tw1_v, bias_v, out_v):
    wid = lax.axis_index("s") * 2 + lax.axis_index("c")
    ntok = _RPW * _L
    pltpu.sync_copy(tok_hbm.at[pl.ds(wid * ntok, ntok)], tok_v)
    pltpu.sync_copy(tw0_hbm, tw0_v)
    pltpu.sync_copy(tw1_hbm, tw1_v)
    pltpu.sync_copy(bias_hbm, bias_v)
    bv0 = bias_v[pl.ds(0, 16)]
    bv1 = bias_v[pl.ds(16, 16)]
    row_stride = lax.broadcasted_iota(jnp.int32, (16,), 0) * _L

    def body(l, carry):
        new = []
        for g in range(_GPW):
            a0, a1 = carry[2 * g], carry[2 * g + 1]
            tok16 = plsc.load_gather(tok_v, [row_stride + (g * (16 * _L) + l)])
            v0 = plsc.load_gather(tw0_v, [tok16])
            v1 = plsc.load_gather(tw1_v, [tok16])
            new.append(a0 + v0)
            new.append(a1 + v1)
        return tuple(new)

    zero = jnp.zeros((16,), jnp.float32)
    accs = lax.fori_loop(0, _L, body, (zero,) * (2 * _GPW))
    for g in range(_GPW):
        out_v[0, pl.ds(g * 16, 16)] = accs[2 * g] + bv0
        out_v[1, pl.ds(g * 16, 16)] = accs[2 * g + 1] + bv1

    pltpu.sync_copy(out_v, out_hbm.at[wid])


def kernel(text_token, table, W, b):
    tw_t = _fold_table(table, W.T)
    tw0 = tw_t[0]
    tw1 = tw_t[1]
    tok_flat = text_token.reshape(-1)
    bias16 = jnp.broadcast_to(b[:, None], (_OUT, 16)).reshape(-1)
    out = _sc_pool(tok_flat, tw0, tw1, bias16)
    return jnp.transpose(out, (0, 2, 1)).reshape(_B, _OUT)
